# baseline (speedup 1.0000x reference)
import functools

import jax
import jax.numpy as jnp
from jax import lax
from jax.experimental import pallas as pl
from jax.experimental.pallas import tpu as pltpu, tpu_sc as plsc

_info = plsc.get_sparse_core_info()
_NC, _NS = _info.num_cores, _info.num_subcores
_NW = _NC * _NS


def _embed_lookup(city2d, table):
    n_rows = city2d.shape[1]
    v, d = table.shape
    nc = d // 16
    mesh = plsc.VectorSubcoreMesh(core_axis_name="c", subcore_axis_name="s")

    @functools.partial(
        pl.kernel,
        mesh=mesh,
        out_type=jax.ShapeDtypeStruct((_NW * n_rows, d), jnp.float32),
        scratch_types=[
            pltpu.VMEM((n_rows,), jnp.int32),
            pltpu.VMEM((v, d), jnp.float32),
            pltpu.VMEM((v * d,), jnp.float32),
            pltpu.VMEM((n_rows, d), jnp.float32),
            pltpu.SemaphoreType.DMA,
            pltpu.SemaphoreType.DMA,
        ],
        compiler_params=pltpu.CompilerParams(needs_layout_passes=False),
    )
    def k(tab_hbm, idx_hbm, out_hbm, idx_v, tab_v, tabf_v, rows_v, isem, osem):
        wid = lax.axis_index("s") * _NC + lax.axis_index("c")
        ld_tab = pltpu.async_copy(tab_hbm, tab_v, isem)
        ld_idx = pltpu.async_copy(idx_hbm.at[wid], idx_v, isem)
        ld_tab.wait()
        ld_idx.wait()

        for r in range(v):
            for c in range(nc):
                tabf_v[pl.ds(r * d + c * 16, 16)] = tab_v[r, pl.ds(c * 16, 16)]
        for g in range(n_rows // 16):
            sl = pl.ds(g * 16, 16)
            idx_v[sl] = idx_v[sl] * d

        cols = [lax.iota(jnp.int32, 16) + c * 16 for c in range(nc)]
        @plsc.parallel_loop(0, n_rows, unroll=16)
        def body(i):
            base = plsc.load_gather(idx_v, [jnp.full((16,), i, jnp.int32)])
            for c in range(nc):
                rows_v[i, pl.ds(c * 16, 16)] = plsc.load_gather(
                    tabf_v, [base + cols[c]]
                )

        pltpu.sync_copy(rows_v, out_hbm.at[pl.ds(wid * n_rows, n_rows)])

    return k(table, city2d)


def kernel(city, table):
    b = city.shape[0]
    city2d = city.astype(jnp.int32).reshape(_NW, b // _NW)
    return _embed_lookup(city2d, table)


# speedup vs baseline: 1.0099x; 1.0099x over previous
import functools

import jax
import jax.numpy as jnp
from jax import lax
from jax.experimental import pallas as pl
from jax.experimental.pallas import tpu as pltpu, tpu_sc as plsc

_info = plsc.get_sparse_core_info()
_NC, _NS = _info.num_cores, _info.num_subcores
_NW = _NC * _NS


def _embed_lookup(city2d, table):
    n_rows = city2d.shape[1]
    v, d = table.shape
    nc = d // 16
    mesh = plsc.VectorSubcoreMesh(core_axis_name="c", subcore_axis_name="s")

    @functools.partial(
        pl.kernel,
        mesh=mesh,
        out_type=jax.ShapeDtypeStruct((_NW * n_rows, d), jnp.float32),
        scratch_types=[
            pltpu.VMEM((n_rows,), jnp.int32),
            pltpu.VMEM((v, d), jnp.float32),
            pltpu.VMEM((n_rows, d), jnp.float32),
            pltpu.SemaphoreType.DMA,
        ],
        compiler_params=pltpu.CompilerParams(needs_layout_passes=False),
    )
    def k(tab_hbm, idx_hbm, out_hbm, idx_v, tab_v, rows_v, isem):
        wid = lax.axis_index("s") * _NC + lax.axis_index("c")
        ld_tab = pltpu.async_copy(tab_hbm, tab_v, isem)
        ld_idx = pltpu.async_copy(idx_hbm.at[wid], idx_v, isem)
        ld_tab.wait()
        ld_idx.wait()

        cols = [lax.iota(jnp.int32, 16) + c * 16 for c in range(nc)]

        @plsc.parallel_loop(0, n_rows, unroll=16)
        def body(i):
            row_splat = plsc.load_gather(idx_v, [jnp.full((16,), i, jnp.int32)])
            for c in range(nc):
                rows_v[i, pl.ds(c * 16, 16)] = plsc.load_gather(
                    tab_v, [row_splat, cols[c]]
                )

        pltpu.sync_copy(rows_v, out_hbm.at[pl.ds(wid * n_rows, n_rows)])

    return k(table, city2d)


def kernel(city, table):
    b = city.shape[0]
    city2d = city.astype(jnp.int32).reshape(_NW, b // _NW)
    return _embed_lookup(city2d, table)
